# baseline (speedup 1.0000x reference)
import functools

import jax
import jax.numpy as jnp
from jax import lax
from jax.experimental import pallas as pl
from jax.experimental.pallas import tpu as pltpu
from jax.experimental.pallas import tpu_sc as plsc

N = 10000
E = 320000
NSUB = 16
EPS = E // NSUB
BLK = 4000
NBLK = EPS // BLK
N_PAD = 10112
SLICE = N_PAD // NSUB
W = 8

_f32 = jnp.float32


def _proj(h, w_ref, b_ref):
    o = jnp.dot(h, w_ref[...], preferred_element_type=_f32)
    return jnp.maximum(o + b_ref[...], 0.0)


def _pad_w(h5):
    return jnp.pad(h5, ((0, 0), (0, W - 5)))


def _tc1_body(x_ref, w2_ref, b2_ref, w3_ref, b3_ref, w4_ref, b4_ref,
              o2_ref, o3_ref, o4_ref):
    x = x_ref[...]
    o2_ref[...] = _pad_w(_proj(x, w2_ref, b2_ref))
    o3_ref[...] = _pad_w(_proj(x, w3_ref, b3_ref))
    o4_ref[...] = _pad_w(_proj(x, w4_ref, b4_ref))


def _tc2_body(h2_ref, h3_ref, h4_ref,
              wo2_ref, bo2_ref, wo3_ref, bo3_ref, wo4_ref, bo4_ref,
              wi2_ref, bi2_ref, wi3_ref, bi3_ref, wi4_ref, bi4_ref,
              o2_ref, o3_ref, o4_ref):
    feat = jnp.concatenate([_proj(h2_ref[...][:, :5], wo2_ref, bo2_ref),
                            _proj(h3_ref[...][:, :5], wo3_ref, bo3_ref),
                            _proj(h4_ref[...][:, :5], wo4_ref, bo4_ref)], axis=1)
    o2_ref[...] = _pad_w(_proj(feat, wi2_ref, bi2_ref))
    o3_ref[...] = _pad_w(_proj(feat, wi3_ref, bi3_ref))
    o4_ref[...] = _pad_w(_proj(feat, wi4_ref, bi4_ref))


def _tc3_body(h2_ref, h3_ref, h4_ref,
              wo2_ref, bo2_ref, wo3_ref, bo3_ref, wo4_ref, bo4_ref,
              fcw_ref, fcb_ref, out_ref):
    feat = jnp.concatenate([_proj(h2_ref[...][:, :5], wo2_ref, bo2_ref),
                            _proj(h3_ref[...][:, :5], wo3_ref, bo3_ref),
                            _proj(h4_ref[...][:, :5], wo4_ref, bo4_ref)], axis=1)
    res = jnp.dot(feat, fcw_ref[...], preferred_element_type=_f32) + fcb_ref[...]
    out_ref[...] = res[:N]


_MESH = plsc.VectorSubcoreMesh(core_axis_name="c", subcore_axis_name="s")


@functools.partial(
    pl.kernel,
    out_type=(jax.ShapeDtypeStruct((N_PAD, W), _f32),) * 3,
    mesh=_MESH,
    scratch_types=[
        pltpu.VMEM_SHARED((N_PAD, W), _f32),
        pltpu.VMEM_SHARED((N_PAD, W), _f32),
        pltpu.VMEM((NBLK, BLK), jnp.int32),
        pltpu.VMEM((NBLK, BLK), jnp.int32),
        pltpu.VMEM((BLK, W), _f32),
        pltpu.VMEM((BLK, W), _f32),
        pltpu.SemaphoreType.DMA,
        pltpu.SemaphoreType.DMA,
    ],
    compiler_params=pltpu.CompilerParams(use_tc_tiling_on_sc=False),
)
def _mp_kernel(h2_hbm, h3_hbm, h4_hbm,
               e2_hbm, e3_hbm, e4_hbm,
               zeros_hbm,
               out2_hbm, out3_hbm, out4_hbm,
               tab0, tab1, src_v, dst_v, buf0, buf1, sem_g, sem_s):
    cid = lax.axis_index("c")
    sid = lax.axis_index("s")
    blk_lo = sid * NBLK
    tab_lo = sid * SLICE
    bufs = [buf0, buf1]
    tabs = [tab0, tab1]

    def my_tab(ref):
        return ref.at[pl.ds(tab_lo, SLICE)]

    def wait_one(sem):
        pltpu.make_async_copy(zeros_hbm.at[pl.ds(0, BLK)], buf0, sem).wait()

    def run_pattern(h0_hbm, edges_hbm, out_hbm, length):
        pltpu.sync_copy(edges_hbm.at[0].at[pl.ds(blk_lo, NBLK)], src_v)
        pltpu.sync_copy(edges_hbm.at[1].at[pl.ds(blk_lo, NBLK)], dst_v)
        for r in range(length):
            cur = h0_hbm if r == 0 else tabs[(r - 1) % 2]
            acc = tabs[r % 2]
            pltpu.sync_copy(my_tab(zeros_hbm), my_tab(acc))
            plsc.subcore_barrier()

            def gidx(i):
                return src_v.at[i]

            def sidx(i):
                return dst_v.at[i]

            pltpu.async_copy(cur.at[gidx(0)], bufs[0], sem_g)
            for i in range(NBLK):
                wait_one(sem_g)
                pltpu.async_copy(bufs[i % 2], acc.at[sidx(i)], sem_s, add=True)
                if i >= 1:
                    wait_one(sem_s)
                if i + 1 < NBLK:
                    pltpu.async_copy(cur.at[gidx(i + 1)], bufs[(i + 1) % 2], sem_g)
            wait_one(sem_s)
            plsc.subcore_barrier()
        pltpu.sync_copy(my_tab(tabs[(length - 1) % 2]), my_tab(out_hbm))

    @pl.when(cid == 0)
    def _():
        run_pattern(h4_hbm, e4_hbm, out4_hbm, 4)
        plsc.subcore_barrier()
        plsc.subcore_barrier()

    @pl.when(cid == 1)
    def _():
        run_pattern(h2_hbm, e2_hbm, out2_hbm, 2)
        run_pattern(h3_hbm, e3_hbm, out3_hbm, 3)


def _prep_edges(edge_index):
    return edge_index.reshape(2, NSUB * NBLK, BLK)


def kernel(batch, edge_index_c2, edge_index_c3, edge_index_c4, fc_b, fc_w,
           l0_c2_b_in, l0_c2_b_out, l0_c2_w_in, l0_c2_w_out,
           l0_c3_b_in, l0_c3_b_out, l0_c3_w_in, l0_c3_w_out,
           l0_c4_b_in, l0_c4_b_out, l0_c4_w_in, l0_c4_w_out,
           l1_c2_b_in, l1_c2_b_out, l1_c2_w_in, l1_c2_w_out,
           l1_c3_b_in, l1_c3_b_out, l1_c3_w_in, l1_c3_w_out,
           l1_c4_b_in, l1_c4_b_out, l1_c4_w_in, l1_c4_w_out,
           x):
    del batch

    zeros = jnp.zeros((N_PAD, W), _f32)

    e2 = _prep_edges(edge_index_c2)
    e3 = _prep_edges(edge_index_c3)
    e4 = _prep_edges(edge_index_c4)

    h2, h3, h4 = pl.pallas_call(
        _tc1_body, out_shape=[jax.ShapeDtypeStruct((N, W), _f32)] * 3,
    )(x, l0_c2_w_in, l0_c2_b_in.reshape(1, 5),
      l0_c3_w_in, l0_c3_b_in.reshape(1, 5),
      l0_c4_w_in, l0_c4_b_in.reshape(1, 5))

    m2, m3, m4 = _mp_kernel(h2, h3, h4, e2, e3, e4, zeros)

    g2, g3, g4 = pl.pallas_call(
        _tc2_body, out_shape=[jax.ShapeDtypeStruct((N_PAD, W), _f32)] * 3,
    )(m2, m3, m4,
      l0_c2_w_out, l0_c2_b_out.reshape(1, 10),
      l0_c3_w_out, l0_c3_b_out.reshape(1, 10),
      l0_c4_w_out, l0_c4_b_out.reshape(1, 10),
      l1_c2_w_in, l1_c2_b_in.reshape(1, 5),
      l1_c3_w_in, l1_c3_b_in.reshape(1, 5),
      l1_c4_w_in, l1_c4_b_in.reshape(1, 5))

    n2, n3, n4 = _mp_kernel(g2, g3, g4, e2, e3, e4, zeros)

    return pl.pallas_call(
        _tc3_body, out_shape=jax.ShapeDtypeStruct((N, 10), _f32),
    )(n2, n3, n4,
      l1_c2_w_out, l1_c2_b_out.reshape(1, 15),
      l1_c3_w_out, l1_c3_b_out.reshape(1, 15),
      l1_c4_w_out, l1_c4_b_out.reshape(1, 15),
      fc_w, fc_b.reshape(1, 10))


# speedup vs baseline: 1.0432x; 1.0432x over previous
import functools

import jax
import jax.numpy as jnp
from jax import lax
from jax.experimental import pallas as pl
from jax.experimental.pallas import tpu as pltpu
from jax.experimental.pallas import tpu_sc as plsc

N = 10000
E = 320000
NSUB = 16
EPS = E // NSUB
BLK = 2000
NBLK = EPS // BLK
N_PAD = 10112
SLICE = N_PAD // NSUB
W = 8

_f32 = jnp.float32


def _proj(h, w_ref, b_ref):
    o = jnp.dot(h, w_ref[...], preferred_element_type=_f32)
    return jnp.maximum(o + b_ref[...], 0.0)


def _pad_w(h5):
    return jnp.pad(h5, ((0, 0), (0, W - 5)))


def _tc1_body(x_ref, w2_ref, b2_ref, w3_ref, b3_ref, w4_ref, b4_ref,
              o2_ref, o3_ref, o4_ref):
    x = x_ref[...]
    o2_ref[...] = _pad_w(_proj(x, w2_ref, b2_ref))
    o3_ref[...] = _pad_w(_proj(x, w3_ref, b3_ref))
    o4_ref[...] = _pad_w(_proj(x, w4_ref, b4_ref))


def _tc2_body(h2_ref, h3_ref, h4_ref,
              wo2_ref, bo2_ref, wo3_ref, bo3_ref, wo4_ref, bo4_ref,
              wi2_ref, bi2_ref, wi3_ref, bi3_ref, wi4_ref, bi4_ref,
              o2_ref, o3_ref, o4_ref):
    feat = jnp.concatenate([_proj(h2_ref[...][:, :5], wo2_ref, bo2_ref),
                            _proj(h3_ref[...][:, :5], wo3_ref, bo3_ref),
                            _proj(h4_ref[...][:, :5], wo4_ref, bo4_ref)], axis=1)
    o2_ref[...] = _pad_w(_proj(feat, wi2_ref, bi2_ref))
    o3_ref[...] = _pad_w(_proj(feat, wi3_ref, bi3_ref))
    o4_ref[...] = _pad_w(_proj(feat, wi4_ref, bi4_ref))


def _tc3_body(h2_ref, h3_ref, h4_ref,
              wo2_ref, bo2_ref, wo3_ref, bo3_ref, wo4_ref, bo4_ref,
              fcw_ref, fcb_ref, out_ref):
    feat = jnp.concatenate([_proj(h2_ref[...][:, :5], wo2_ref, bo2_ref),
                            _proj(h3_ref[...][:, :5], wo3_ref, bo3_ref),
                            _proj(h4_ref[...][:, :5], wo4_ref, bo4_ref)], axis=1)
    res = jnp.dot(feat, fcw_ref[...], preferred_element_type=_f32) + fcb_ref[...]
    out_ref[...] = res[:N]


_MESH = plsc.VectorSubcoreMesh(core_axis_name="c", subcore_axis_name="s")


@functools.partial(
    pl.kernel,
    out_type=(jax.ShapeDtypeStruct((N_PAD, W), _f32),) * 3,
    mesh=_MESH,
    scratch_types=[
        pltpu.VMEM_SHARED((N_PAD, W), _f32),
        pltpu.VMEM_SHARED((N_PAD, W), _f32),
        pltpu.VMEM_SHARED((N_PAD, W), _f32),
        pltpu.VMEM((NBLK, BLK), jnp.int32),
        pltpu.VMEM((NBLK, BLK), jnp.int32),
        pltpu.VMEM((BLK, W), _f32),
        pltpu.VMEM((BLK, W), _f32),
        pltpu.VMEM((BLK, W), _f32),
        pltpu.VMEM((BLK, W), _f32),
        pltpu.SemaphoreType.DMA,
        pltpu.SemaphoreType.DMA,
        pltpu.SemaphoreType.DMA,
    ],
    compiler_params=pltpu.CompilerParams(use_tc_tiling_on_sc=False),
)
def _mp_kernel(h2_hbm, h3_hbm, h4_hbm,
               e2_hbm, e3_hbm, e4_hbm,
               zeros_hbm,
               out2_hbm, out3_hbm, out4_hbm,
               tab0, tab1, tab2, src_v, dst_v, buf0, buf1, buf2, buf3,
               sem_g, sem_s, sem_z):
    cid = lax.axis_index("c")
    sid = lax.axis_index("s")
    blk_lo = sid * NBLK
    tab_lo = sid * SLICE
    bufs = [buf0, buf1, buf2, buf3]
    tabs = [tab0, tab1, tab2]

    def my_tab(ref):
        return ref.at[pl.ds(tab_lo, SLICE)]

    def wait_one(sem):
        pltpu.make_async_copy(zeros_hbm.at[pl.ds(0, BLK)], buf0, sem).wait()

    def run_pattern(h0_hbm, edges_hbm, out_hbm, length):
        pltpu.sync_copy(edges_hbm.at[0].at[pl.ds(blk_lo, NBLK)], src_v)
        pltpu.sync_copy(edges_hbm.at[1].at[pl.ds(blk_lo, NBLK)], dst_v)
        pltpu.sync_copy(my_tab(zeros_hbm), my_tab(tabs[0]))
        plsc.subcore_barrier()
        for r in range(length):
            cur = h0_hbm if r == 0 else tabs[(r - 1) % 3]
            acc = tabs[r % 3]
            if r + 1 < length:
                pltpu.async_copy(my_tab(zeros_hbm), my_tab(tabs[(r + 1) % 3]), sem_z)

            def gidx(i):
                return src_v.at[i]

            def sidx(i):
                return dst_v.at[i]

            pltpu.async_copy(cur.at[gidx(0)], bufs[0], sem_g)
            pltpu.async_copy(cur.at[gidx(1)], bufs[1], sem_g)

            def group_body(g, _):
                for di in range(4):
                    i = 4 * g + di
                    buf = bufs[di]
                    wait_one(sem_g)
                    pltpu.async_copy(buf, acc.at[sidx(i)], sem_s, add=True)
                    @pl.when(i >= 2)
                    def _():
                        wait_one(sem_s)

                    nxt = bufs[(di + 2) % 4]

                    @pl.when(i + 2 < NBLK)
                    def _():
                        pltpu.async_copy(cur.at[gidx(i + 2)], nxt, sem_g)
                return 0

            lax.fori_loop(0, NBLK // 4, group_body, 0)
            for i in range(4 * (NBLK // 4), NBLK):
                buf = bufs[i % 4]
                wait_one(sem_g)
                pltpu.async_copy(buf, acc.at[sidx(i)], sem_s, add=True)
                wait_one(sem_s)
                if i + 2 < NBLK:
                    pltpu.async_copy(cur.at[gidx(i + 2)], bufs[(i + 2) % 4], sem_g)
            wait_one(sem_s)
            wait_one(sem_s)
            if r + 1 < length:
                pltpu.make_async_copy(my_tab(zeros_hbm), my_tab(tabs[(r + 1) % 3]), sem_z).wait()
            plsc.subcore_barrier()
        pltpu.sync_copy(my_tab(tabs[(length - 1) % 3]), my_tab(out_hbm))

    @pl.when(cid == 0)
    def _():
        run_pattern(h4_hbm, e4_hbm, out4_hbm, 4)
        plsc.subcore_barrier()
        plsc.subcore_barrier()

    @pl.when(cid == 1)
    def _():
        run_pattern(h2_hbm, e2_hbm, out2_hbm, 2)
        run_pattern(h3_hbm, e3_hbm, out3_hbm, 3)


def _prep_edges(edge_index):
    return edge_index.reshape(2, NSUB * NBLK, BLK)


def kernel(batch, edge_index_c2, edge_index_c3, edge_index_c4, fc_b, fc_w,
           l0_c2_b_in, l0_c2_b_out, l0_c2_w_in, l0_c2_w_out,
           l0_c3_b_in, l0_c3_b_out, l0_c3_w_in, l0_c3_w_out,
           l0_c4_b_in, l0_c4_b_out, l0_c4_w_in, l0_c4_w_out,
           l1_c2_b_in, l1_c2_b_out, l1_c2_w_in, l1_c2_w_out,
           l1_c3_b_in, l1_c3_b_out, l1_c3_w_in, l1_c3_w_out,
           l1_c4_b_in, l1_c4_b_out, l1_c4_w_in, l1_c4_w_out,
           x):
    del batch

    zeros = jnp.zeros((N_PAD, W), _f32)

    e2 = _prep_edges(edge_index_c2)
    e3 = _prep_edges(edge_index_c3)
    e4 = _prep_edges(edge_index_c4)

    h2, h3, h4 = pl.pallas_call(
        _tc1_body, out_shape=[jax.ShapeDtypeStruct((N, W), _f32)] * 3,
    )(x, l0_c2_w_in, l0_c2_b_in.reshape(1, 5),
      l0_c3_w_in, l0_c3_b_in.reshape(1, 5),
      l0_c4_w_in, l0_c4_b_in.reshape(1, 5))

    m2, m3, m4 = _mp_kernel(h2, h3, h4, e2, e3, e4, zeros)

    g2, g3, g4 = pl.pallas_call(
        _tc2_body, out_shape=[jax.ShapeDtypeStruct((N_PAD, W), _f32)] * 3,
    )(m2, m3, m4,
      l0_c2_w_out, l0_c2_b_out.reshape(1, 10),
      l0_c3_w_out, l0_c3_b_out.reshape(1, 10),
      l0_c4_w_out, l0_c4_b_out.reshape(1, 10),
      l1_c2_w_in, l1_c2_b_in.reshape(1, 5),
      l1_c3_w_in, l1_c3_b_in.reshape(1, 5),
      l1_c4_w_in, l1_c4_b_in.reshape(1, 5))

    n2, n3, n4 = _mp_kernel(g2, g3, g4, e2, e3, e4, zeros)

    return pl.pallas_call(
        _tc3_body, out_shape=jax.ShapeDtypeStruct((N, 10), _f32),
    )(n2, n3, n4,
      l1_c2_w_out, l1_c2_b_out.reshape(1, 15),
      l1_c3_w_out, l1_c3_b_out.reshape(1, 15),
      l1_c4_w_out, l1_c4_b_out.reshape(1, 15),
      fc_w, fc_b.reshape(1, 10))


# speedup vs baseline: 1.2466x; 1.1950x over previous
import functools

import jax
import jax.numpy as jnp
from jax import lax
from jax.experimental import pallas as pl
from jax.experimental.pallas import tpu as pltpu
from jax.experimental.pallas import tpu_sc as plsc

N = 10000
E = 320000
NSUB = 16
EPS = E // NSUB
BLK = 2000
NBLK = EPS // BLK
N_PAD = 10112
SLICE = N_PAD // NSUB
W = 8

_f32 = jnp.float32


def _proj(h, w_ref, b_ref):
    o = jnp.dot(h, w_ref[...], preferred_element_type=_f32)
    return jnp.maximum(o + b_ref[...], 0.0)


def _pad_w(h5):
    return jnp.pad(h5, ((0, 0), (0, W - 5)))


def _tc1_body(x_ref, w2_ref, b2_ref, w3_ref, b3_ref, w4_ref, b4_ref,
              o2_ref, o3_ref, o4_ref):
    x = x_ref[...]
    o2_ref[...] = _pad_w(_proj(x, w2_ref, b2_ref))
    o3_ref[...] = _pad_w(_proj(x, w3_ref, b3_ref))
    o4_ref[...] = _pad_w(_proj(x, w4_ref, b4_ref))


def _tc2_body(h2_ref, h3_ref, h4_ref,
              wo2_ref, bo2_ref, wo3_ref, bo3_ref, wo4_ref, bo4_ref,
              wi2_ref, bi2_ref, wi3_ref, bi3_ref, wi4_ref, bi4_ref,
              o2_ref, o3_ref, o4_ref):
    feat = jnp.concatenate([_proj(h2_ref[...][:, :5], wo2_ref, bo2_ref),
                            _proj(h3_ref[...][:, :5], wo3_ref, bo3_ref),
                            _proj(h4_ref[...][:, :5], wo4_ref, bo4_ref)], axis=1)
    o2_ref[...] = _pad_w(_proj(feat, wi2_ref, bi2_ref))
    o3_ref[...] = _pad_w(_proj(feat, wi3_ref, bi3_ref))
    o4_ref[...] = _pad_w(_proj(feat, wi4_ref, bi4_ref))


def _tc3_body(h2_ref, h3_ref, h4_ref,
              wo2_ref, bo2_ref, wo3_ref, bo3_ref, wo4_ref, bo4_ref,
              fcw_ref, fcb_ref, out_ref):
    feat = jnp.concatenate([_proj(h2_ref[...][:, :5], wo2_ref, bo2_ref),
                            _proj(h3_ref[...][:, :5], wo3_ref, bo3_ref),
                            _proj(h4_ref[...][:, :5], wo4_ref, bo4_ref)], axis=1)
    res = jnp.dot(feat, fcw_ref[...], preferred_element_type=_f32) + fcb_ref[...]
    out_ref[...] = res[:N]


_MESH = plsc.VectorSubcoreMesh(core_axis_name="c", subcore_axis_name="s")


@functools.partial(
    pl.kernel,
    out_type=(jax.ShapeDtypeStruct((N_PAD, W), _f32),) * 3,
    mesh=_MESH,
    scratch_types=[
        pltpu.VMEM_SHARED((N_PAD, W), _f32),
        pltpu.VMEM_SHARED((N_PAD, W), _f32),
        pltpu.VMEM_SHARED((N_PAD, W), _f32),
        pltpu.VMEM((NBLK, BLK), jnp.int32),
        pltpu.VMEM((NBLK, BLK), jnp.int32),
        pltpu.VMEM((BLK, W), _f32),
        pltpu.VMEM((BLK, W), _f32),
        pltpu.VMEM((BLK, W), _f32),
        pltpu.VMEM((BLK, W), _f32),
        pltpu.SemaphoreType.DMA,
        pltpu.SemaphoreType.DMA,
        pltpu.SemaphoreType.DMA,
    ],
    compiler_params=pltpu.CompilerParams(use_tc_tiling_on_sc=False),
)
def _mp_kernel(h2_hbm, h3_hbm, h4_hbm,
               e2_hbm, e3_hbm, e4_hbm,
               zeros_hbm,
               out2_hbm, out3_hbm, out4_hbm,
               tab0, tab1, tab2, src_v, dst_v, buf0, buf1, buf2, buf3,
               sem_g, sem_s, sem_z):
    cid = lax.axis_index("c")
    sid = lax.axis_index("s")
    blk_lo = sid * NBLK
    tab_lo = sid * SLICE
    bufs = [buf0, buf1, buf2, buf3]
    tabs = [tab0, tab1, tab2]

    def my_tab(ref):
        return ref.at[pl.ds(tab_lo, SLICE)]

    def wait_one(sem):
        pltpu.make_async_copy(zeros_hbm.at[pl.ds(0, BLK)], buf0, sem).wait()

    def run_pattern(h0_hbm, edges_hbm, out_hbm, length):
        pltpu.sync_copy(edges_hbm.at[0].at[pl.ds(blk_lo, NBLK)], src_v)
        pltpu.sync_copy(edges_hbm.at[1].at[pl.ds(blk_lo, NBLK)], dst_v)
        pltpu.async_copy(h0_hbm.at[pl.ds(sid * 625, 625)],
                         tabs[2].at[pl.ds(sid * 625, 625)], sem_z)
        pltpu.sync_copy(my_tab(zeros_hbm), my_tab(tabs[0]))
        pltpu.make_async_copy(h0_hbm.at[pl.ds(sid * 625, 625)],
                              tabs[2].at[pl.ds(sid * 625, 625)], sem_z).wait()
        plsc.subcore_barrier()
        for r in range(length):
            cur = tabs[(r - 1) % 3]
            acc = tabs[r % 3]
            if r + 1 < length:
                pltpu.async_copy(my_tab(zeros_hbm), my_tab(tabs[(r + 1) % 3]), sem_z)

            def gidx(i):
                return src_v.at[i]

            def sidx(i):
                return dst_v.at[i]

            pltpu.async_copy(cur.at[gidx(0)], bufs[0], sem_g)
            pltpu.async_copy(cur.at[gidx(1)], bufs[1], sem_g)

            def group_body(g, _):
                for di in range(4):
                    i = 4 * g + di
                    buf = bufs[di]
                    wait_one(sem_g)
                    pltpu.async_copy(buf, acc.at[sidx(i)], sem_s, add=True)
                    @pl.when(i >= 2)
                    def _():
                        wait_one(sem_s)

                    nxt = bufs[(di + 2) % 4]

                    @pl.when(i + 2 < NBLK)
                    def _():
                        pltpu.async_copy(cur.at[gidx(i + 2)], nxt, sem_g)
                return 0

            lax.fori_loop(0, NBLK // 4, group_body, 0)
            for i in range(4 * (NBLK // 4), NBLK):
                buf = bufs[i % 4]
                wait_one(sem_g)
                pltpu.async_copy(buf, acc.at[sidx(i)], sem_s, add=True)
                wait_one(sem_s)
                if i + 2 < NBLK:
                    pltpu.async_copy(cur.at[gidx(i + 2)], bufs[(i + 2) % 4], sem_g)
            wait_one(sem_s)
            wait_one(sem_s)
            if r + 1 < length:
                pltpu.make_async_copy(my_tab(zeros_hbm), my_tab(tabs[(r + 1) % 3]), sem_z).wait()
            plsc.subcore_barrier()
        pltpu.sync_copy(my_tab(tabs[(length - 1) % 3]), my_tab(out_hbm))

    @pl.when(cid == 0)
    def _():
        run_pattern(h4_hbm, e4_hbm, out4_hbm, 4)
        plsc.subcore_barrier()
        plsc.subcore_barrier()

    @pl.when(cid == 1)
    def _():
        run_pattern(h2_hbm, e2_hbm, out2_hbm, 2)
        run_pattern(h3_hbm, e3_hbm, out3_hbm, 3)


def _prep_edges(edge_index):
    return edge_index.reshape(2, NSUB * NBLK, BLK)


def kernel(batch, edge_index_c2, edge_index_c3, edge_index_c4, fc_b, fc_w,
           l0_c2_b_in, l0_c2_b_out, l0_c2_w_in, l0_c2_w_out,
           l0_c3_b_in, l0_c3_b_out, l0_c3_w_in, l0_c3_w_out,
           l0_c4_b_in, l0_c4_b_out, l0_c4_w_in, l0_c4_w_out,
           l1_c2_b_in, l1_c2_b_out, l1_c2_w_in, l1_c2_w_out,
           l1_c3_b_in, l1_c3_b_out, l1_c3_w_in, l1_c3_w_out,
           l1_c4_b_in, l1_c4_b_out, l1_c4_w_in, l1_c4_w_out,
           x):
    del batch

    zeros = jnp.zeros((N_PAD, W), _f32)

    e2 = _prep_edges(edge_index_c2)
    e3 = _prep_edges(edge_index_c3)
    e4 = _prep_edges(edge_index_c4)

    h2, h3, h4 = pl.pallas_call(
        _tc1_body, out_shape=[jax.ShapeDtypeStruct((N, W), _f32)] * 3,
    )(x, l0_c2_w_in, l0_c2_b_in.reshape(1, 5),
      l0_c3_w_in, l0_c3_b_in.reshape(1, 5),
      l0_c4_w_in, l0_c4_b_in.reshape(1, 5))

    m2, m3, m4 = _mp_kernel(h2, h3, h4, e2, e3, e4, zeros)

    g2, g3, g4 = pl.pallas_call(
        _tc2_body, out_shape=[jax.ShapeDtypeStruct((N_PAD, W), _f32)] * 3,
    )(m2, m3, m4,
      l0_c2_w_out, l0_c2_b_out.reshape(1, 10),
      l0_c3_w_out, l0_c3_b_out.reshape(1, 10),
      l0_c4_w_out, l0_c4_b_out.reshape(1, 10),
      l1_c2_w_in, l1_c2_b_in.reshape(1, 5),
      l1_c3_w_in, l1_c3_b_in.reshape(1, 5),
      l1_c4_w_in, l1_c4_b_in.reshape(1, 5))

    n2, n3, n4 = _mp_kernel(g2, g3, g4, e2, e3, e4, zeros)

    return pl.pallas_call(
        _tc3_body, out_shape=jax.ShapeDtypeStruct((N, 10), _f32),
    )(n2, n3, n4,
      l1_c2_w_out, l1_c2_b_out.reshape(1, 15),
      l1_c3_w_out, l1_c3_b_out.reshape(1, 15),
      l1_c4_w_out, l1_c4_b_out.reshape(1, 15),
      fc_w, fc_b.reshape(1, 10))
